# baseline (speedup 1.0000x reference)
import functools

import jax
import jax.numpy as jnp
from jax import lax
from jax.experimental import pallas as pl
from jax.experimental.pallas import tpu as pltpu
from jax.experimental.pallas import tpu_sc as plsc

EMBED_DIM = 64
BATCH = 16384

_NC = 2
_NS = 16
_NW = _NC * _NS
_BPW = BATCH // _NW


def _gather_body(ut, it, uidx, iidx, uout, iout,
                 uidx_v, iidx_v, urows_v, irows_v, usem, isem):
    wid = lax.axis_index("s") * _NC + lax.axis_index("c")
    base = wid * _BPW
    pltpu.sync_copy(uidx.at[pl.ds(base, _BPW)], uidx_v)
    pltpu.sync_copy(iidx.at[pl.ds(base, _BPW)], iidx_v)
    cu = pltpu.async_copy(ut.at[uidx_v], urows_v, usem)
    ci = pltpu.async_copy(it.at[iidx_v], irows_v, isem)
    cu.wait()
    pltpu.sync_copy(urows_v, uout.at[pl.ds(base, _BPW)])
    ci.wait()
    pltpu.sync_copy(irows_v, iout.at[pl.ds(base, _BPW)])


@jax.jit
def _gather(user_table, item_table, user_idx, item_idx):
    mesh = plsc.VectorSubcoreMesh(core_axis_name="c", subcore_axis_name="s")
    emb = jax.ShapeDtypeStruct((BATCH, EMBED_DIM), jnp.float32)
    run = pl.kernel(
        _gather_body,
        mesh=mesh,
        out_type=(emb, emb),
        scratch_types=[
            pltpu.VMEM((_BPW,), jnp.int32),
            pltpu.VMEM((_BPW,), jnp.int32),
            pltpu.VMEM((_BPW, EMBED_DIM), jnp.float32),
            pltpu.VMEM((_BPW, EMBED_DIM), jnp.float32),
            pltpu.SemaphoreType.DMA,
            pltpu.SemaphoreType.DMA,
        ],
        compiler_params=pltpu.CompilerParams(use_tc_tiling_on_sc=False),
    )
    return run(user_table, item_table, user_idx, item_idx)


def _mlp_body(u_ref, i_ref, f_ref, w1u_ref, w1i_ref, w1f_ref, b1_ref,
              w2_ref, b2_ref, o_ref):
    h = jnp.dot(u_ref[...], w1u_ref[...], preferred_element_type=jnp.float32)
    h += jnp.dot(i_ref[...], w1i_ref[...], preferred_element_type=jnp.float32)
    f = f_ref[...]
    h += f[:, 0:1] * w1f_ref[0:1, :] + f[:, 1:2] * w1f_ref[1:2, :]
    h = jnp.maximum(h + b1_ref[...], 0.0)
    z = jnp.dot(h, w2_ref[...], preferred_element_type=jnp.float32)
    o_ref[...] = jax.nn.sigmoid(z + b2_ref[...])


_MLP_BLOCK = 2048


@jax.jit
def _mlp(u_emb, i_emb, features, W1u, W1i, W1f, b1, W2, b2):
    nblk = BATCH // _MLP_BLOCK
    batch_spec = lambda w: pl.BlockSpec((_MLP_BLOCK, w), lambda b: (b, 0))
    full_spec = lambda s: pl.BlockSpec(s, lambda b: (0,) * len(s))
    return pl.pallas_call(
        _mlp_body,
        grid=(nblk,),
        in_specs=[
            batch_spec(EMBED_DIM),
            batch_spec(EMBED_DIM),
            batch_spec(2),
            full_spec((EMBED_DIM, EMBED_DIM)),
            full_spec((EMBED_DIM, EMBED_DIM)),
            full_spec((2, EMBED_DIM)),
            full_spec((1, EMBED_DIM)),
            full_spec((EMBED_DIM, 1)),
            full_spec((1, 1)),
        ],
        out_specs=batch_spec(1),
        out_shape=jax.ShapeDtypeStruct((BATCH, 1), jnp.float32),
    )(u_emb, i_emb, features, W1u, W1i, W1f, b1, W2, b2)


def kernel(user_idx, item_idx, features, user_table, item_table, W1, b1, W2, b2):
    u_emb, i_emb = _gather(user_table, item_table,
                           user_idx.astype(jnp.int32), item_idx.astype(jnp.int32))
    W1u = W1[:EMBED_DIM]
    W1i = W1[EMBED_DIM:2 * EMBED_DIM]
    W1f = W1[2 * EMBED_DIM:]
    return _mlp(u_emb, i_emb, features, W1u, W1i, W1f,
                b1.reshape(1, EMBED_DIM), W2, b2.reshape(1, 1))


# speedup vs baseline: 1.5696x; 1.5696x over previous
import functools

import jax
import jax.numpy as jnp
from jax import lax
from jax.experimental import pallas as pl
from jax.experimental.pallas import tpu as pltpu
from jax.experimental.pallas import tpu_sc as plsc

EMBED_DIM = 64
BATCH = 16384

_NC = 2
_NS = 16
_NW = _NC * _NS
_BPW = BATCH // _NW


_LANES = 16


def _do_table(table, idx_v, out_hbm, base, rows_v, sem):
    ngroups = _BPW // _LANES
    lane = lax.iota(jnp.int32, _LANES)

    def fire_group(g, carry):
        v = idx_v[pl.ds(g * _LANES, _LANES)]
        for l in range(_LANES):
            r = lax.reduce_max(jnp.where(lane == l, v, 0), axes=(0,))
            pltpu.make_async_copy(
                table.at[pl.ds(r, 1)],
                rows_v.at[pl.ds(g * _LANES + l, 1)],
                sem,
            ).start()
        return carry

    def wait_group(g, carry):
        for l in range(_LANES):
            pltpu.make_async_copy(
                table.at[pl.ds(0, 1)],
                rows_v.at[pl.ds(g * _LANES + l, 1)],
                sem,
            ).wait()
        return carry

    lax.fori_loop(0, ngroups, fire_group, 0)
    lax.fori_loop(0, ngroups, wait_group, 0)
    pltpu.sync_copy(rows_v, out_hbm.at[pl.ds(base, _BPW)])


def _gather_body(ut, it, uidx, iidx, uout, iout, uidx_v, iidx_v, rows_v, sem):
    wid = lax.axis_index("s") * _NC + lax.axis_index("c")
    base = wid * _BPW
    pltpu.sync_copy(uidx.at[pl.ds(base, _BPW)], uidx_v)
    pltpu.sync_copy(iidx.at[pl.ds(base, _BPW)], iidx_v)
    _do_table(ut, uidx_v, uout, base, rows_v, sem)
    _do_table(it, iidx_v, iout, base, rows_v, sem)


@jax.jit
def _gather(user_table, item_table, user_idx, item_idx):
    mesh = plsc.VectorSubcoreMesh(core_axis_name="c", subcore_axis_name="s")
    emb = jax.ShapeDtypeStruct((BATCH, EMBED_DIM), jnp.float32)
    run = pl.kernel(
        _gather_body,
        mesh=mesh,
        out_type=(emb, emb),
        scratch_types=[
            pltpu.VMEM((_BPW,), jnp.int32),
            pltpu.VMEM((_BPW,), jnp.int32),
            pltpu.VMEM((_BPW, EMBED_DIM), jnp.float32),
            pltpu.SemaphoreType.DMA,
        ],
        compiler_params=pltpu.CompilerParams(needs_layout_passes=False),
    )
    return run(user_table, item_table, user_idx, item_idx)


def _mlp_body(u_ref, i_ref, f_ref, w1u_ref, w1i_ref, w1f_ref, b1_ref,
              w2_ref, b2_ref, o_ref):
    h = jnp.dot(u_ref[...], w1u_ref[...], preferred_element_type=jnp.float32)
    h += jnp.dot(i_ref[...], w1i_ref[...], preferred_element_type=jnp.float32)
    f = f_ref[...]
    h += f[:, 0:1] * w1f_ref[0:1, :] + f[:, 1:2] * w1f_ref[1:2, :]
    h = jnp.maximum(h + b1_ref[...], 0.0)
    z = jnp.dot(h, w2_ref[...], preferred_element_type=jnp.float32)
    o_ref[...] = jax.nn.sigmoid(z + b2_ref[...])


_MLP_BLOCK = 2048


@jax.jit
def _mlp(u_emb, i_emb, features, W1u, W1i, W1f, b1, W2, b2):
    nblk = BATCH // _MLP_BLOCK
    batch_spec = lambda w: pl.BlockSpec((_MLP_BLOCK, w), lambda b: (b, 0))
    full_spec = lambda s: pl.BlockSpec(s, lambda b: (0,) * len(s))
    return pl.pallas_call(
        _mlp_body,
        grid=(nblk,),
        in_specs=[
            batch_spec(EMBED_DIM),
            batch_spec(EMBED_DIM),
            batch_spec(2),
            full_spec((EMBED_DIM, EMBED_DIM)),
            full_spec((EMBED_DIM, EMBED_DIM)),
            full_spec((2, EMBED_DIM)),
            full_spec((1, EMBED_DIM)),
            full_spec((EMBED_DIM, 1)),
            full_spec((1, 1)),
        ],
        out_specs=batch_spec(1),
        out_shape=jax.ShapeDtypeStruct((BATCH, 1), jnp.float32),
    )(u_emb, i_emb, features, W1u, W1i, W1f, b1, W2, b2)


def kernel(user_idx, item_idx, features, user_table, item_table, W1, b1, W2, b2):
    u_emb, i_emb = _gather(user_table, item_table,
                           user_idx.astype(jnp.int32), item_idx.astype(jnp.int32))
    W1u = W1[:EMBED_DIM]
    W1i = W1[EMBED_DIM:2 * EMBED_DIM]
    W1f = W1[2 * EMBED_DIM:]
    return _mlp(u_emb, i_emb, features, W1u, W1i, W1f,
                b1.reshape(1, EMBED_DIM), W2, b2.reshape(1, 1))
